# Initial kernel scaffold; baseline (speedup 1.0000x reference)
import jax
import jax.numpy as jnp
from jax.experimental import pallas as pl


def kernel(x, edge_index, relations, W1, b1, g1, be1, W2, b2, g2, be2, W3, b3):
    raise NotImplementedError("write your pallas kernel here")


# speedup vs baseline: 6.8844x; 6.8844x over previous
import functools

import jax
import jax.numpy as jnp
from jax import lax
from jax.experimental import pallas as pl
from jax.experimental.pallas import tpu as pltpu
from jax.experimental.pallas import tpu_sc as plsc

N = 10000
D = 128
E = 320000

NC = 2
NS = 16
NW = NC * NS
CH = 128
GPT = 80
HGPT = GPT // 2
EPT = CH * GPT
E_PAD = NW * EPT
NZ = 10240
RPT = NZ // NS
DUMMY = N

BR = 1000
BN_S = (1.0 + 1e-5) ** -0.5


def _deg_body(dstr_hbm, zeros128_hbm, ones128_hbm, hist_out, hist_sh, idx_d,
              ones_v):
    c = lax.axis_index("c")
    s = lax.axis_index("s")
    wid = s * NC + c
    r0 = s * RPT
    pltpu.sync_copy(zeros128_hbm, hist_sh.at[pl.ds(r0, RPT)])
    pltpu.sync_copy(ones128_hbm, ones_v)
    pltpu.sync_copy(dstr_hbm.at[wid], idx_d)
    plsc.subcore_barrier()

    def step(g, carry):
        pltpu.sync_copy(ones_v, hist_sh.at[idx_d.at[g // HGPT, g % HGPT]],
                        add=True)
        return carry

    lax.fori_loop(0, GPT, step, 0)
    plsc.subcore_barrier()
    pltpu.sync_copy(hist_sh.at[pl.ds(r0, RPT)],
                    hist_out.at[c, pl.ds(r0, RPT)])


def _sc_deg(dstr, zeros128, ones128):
    mesh = plsc.VectorSubcoreMesh(core_axis_name="c", subcore_axis_name="s")
    fn = pl.kernel(
        _deg_body,
        mesh=mesh,
        out_type=jax.ShapeDtypeStruct((NC, NZ, D), jnp.float32),
        scratch_types=[
            pltpu.VMEM_SHARED((NZ, D), jnp.float32),
            pltpu.VMEM((2, HGPT, CH), jnp.int32),
            pltpu.VMEM((CH, D), jnp.float32),
        ],
    )
    return fn(dstr, zeros128, ones128)


def _agg_body(y_hbm, srcr_hbm, dstr_hbm, zeros128_hbm, z_out, z_sh, idx_s,
              idx_d, rows0, rows1, sem0, sem1):
    c = lax.axis_index("c")
    s = lax.axis_index("s")
    wid = s * NC + c
    r0 = s * RPT
    pltpu.sync_copy(zeros128_hbm, z_sh.at[pl.ds(r0, RPT)])
    plsc.subcore_barrier()

    for h in range(2):
        pltpu.sync_copy(srcr_hbm.at[wid, h], idx_s)
        pltpu.sync_copy(dstr_hbm.at[wid, h], idx_d)
        pltpu.async_copy(y_hbm.at[idx_s.at[0]], rows0, sem0)

        def pair(i, carry):
            g = 2 * i
            pltpu.async_copy(y_hbm.at[idx_s.at[g + 1]], rows1, sem1)
            pltpu.make_async_copy(y_hbm.at[idx_s.at[g]], rows0, sem0).wait()
            pltpu.sync_copy(rows0, z_sh.at[idx_d.at[g]], add=True)

            @pl.when(g + 2 < HGPT)
            def _():
                pltpu.async_copy(y_hbm.at[idx_s.at[g + 2]], rows0, sem0)

            pltpu.make_async_copy(y_hbm.at[idx_s.at[g + 1]], rows1,
                                  sem1).wait()
            pltpu.sync_copy(rows1, z_sh.at[idx_d.at[g + 1]], add=True)
            return carry

        lax.fori_loop(0, HGPT // 2, pair, 0)

    plsc.subcore_barrier()
    pltpu.sync_copy(z_sh.at[pl.ds(r0, RPT)], z_out.at[c, pl.ds(r0, RPT)])


def _sc_agg(y, srcr, dstr, zeros128):
    mesh = plsc.VectorSubcoreMesh(core_axis_name="c", subcore_axis_name="s")
    fn = pl.kernel(
        _agg_body,
        mesh=mesh,
        out_type=jax.ShapeDtypeStruct((NC, NZ, D), jnp.float32),
        scratch_types=[
            pltpu.VMEM_SHARED((NZ, D), jnp.float32),
            pltpu.VMEM((HGPT, CH), jnp.int32),
            pltpu.VMEM((HGPT, CH), jnp.int32),
            pltpu.VMEM((CH, D), jnp.float32),
            pltpu.VMEM((CH, D), jnp.float32),
            pltpu.SemaphoreType.DMA,
            pltpu.SemaphoreType.DMA,
        ],
    )
    return fn(y, srcr, dstr, zeros128)


def _mm_body(x_ref, w_ref, o_ref):
    o_ref[...] = jnp.dot(x_ref[...], w_ref[...],
                         preferred_element_type=jnp.float32)


def _tc_mm(x, w):
    return pl.pallas_call(
        _mm_body,
        grid=(N // BR,),
        in_specs=[
            pl.BlockSpec((BR, D), lambda i: (i, 0)),
            pl.BlockSpec((D, D), lambda i: (0, 0)),
        ],
        out_specs=pl.BlockSpec((BR, D), lambda i: (i, 0)),
        out_shape=jax.ShapeDtypeStruct((N, D), jnp.float32),
    )(x, w)


def _prep_body(hist_ref, xw_ref, dinv_ref, y_ref):
    deg = hist_ref[0, :, 0:1] + hist_ref[1, :, 0:1] + 1.0
    dv = lax.rsqrt(deg)
    dinv_ref[...] = jnp.broadcast_to(dv, (BR, 16))
    y_ref[...] = xw_ref[...] * dv


def _tc_prep(hist, xw):
    return pl.pallas_call(
        _prep_body,
        grid=(N // BR,),
        in_specs=[
            pl.BlockSpec((NC, BR, D), lambda i: (0, i, 0)),
            pl.BlockSpec((BR, D), lambda i: (i, 0)),
        ],
        out_specs=[
            pl.BlockSpec((BR, 16), lambda i: (i, 0)),
            pl.BlockSpec((BR, D), lambda i: (i, 0)),
        ],
        out_shape=[
            jax.ShapeDtypeStruct((N, 16), jnp.float32),
            jax.ShapeDtypeStruct((N, D), jnp.float32),
        ],
    )(hist, xw)


def _stage_body(z_ref, y_ref, dinv_ref, w_ref, b_ref, g_ref, be_ref, yo_ref):
    dv = dinv_ref[:, 0:1]
    t = (z_ref[0] + z_ref[1] + y_ref[...]) * dv + b_ref[...]
    h = jnp.maximum(t * (BN_S * g_ref[...]) + be_ref[...], 0.0)
    yo_ref[...] = jnp.dot(h, w_ref[...],
                          preferred_element_type=jnp.float32) * dv


def _tc_stage(z, y, dinv16, w_next, b, g, be):
    return pl.pallas_call(
        _stage_body,
        grid=(N // BR,),
        in_specs=[
            pl.BlockSpec((NC, BR, D), lambda i: (0, i, 0)),
            pl.BlockSpec((BR, D), lambda i: (i, 0)),
            pl.BlockSpec((BR, 16), lambda i: (i, 0)),
            pl.BlockSpec((D, D), lambda i: (0, 0)),
            pl.BlockSpec((1, D), lambda i: (0, 0)),
            pl.BlockSpec((1, D), lambda i: (0, 0)),
            pl.BlockSpec((1, D), lambda i: (0, 0)),
        ],
        out_specs=pl.BlockSpec((BR, D), lambda i: (i, 0)),
        out_shape=jax.ShapeDtypeStruct((N, D), jnp.float32),
    )(z, y, dinv16, w_next, b.reshape(1, D), g.reshape(1, D),
      be.reshape(1, D))


def _final_body(z_ref, y_ref, dinv_ref, b_ref, o_ref):
    dv = dinv_ref[:, 0:1]
    o = (z_ref[0] + z_ref[1] + y_ref[...]) * dv + b_ref[...]
    m = jnp.max(o, axis=1, keepdims=True)
    lse = jnp.log(jnp.sum(jnp.exp(o - m), axis=1, keepdims=True)) + m
    o_ref[...] = o - lse


def _tc_final(z, y, dinv16, b):
    return pl.pallas_call(
        _final_body,
        grid=(N // BR,),
        in_specs=[
            pl.BlockSpec((NC, BR, D), lambda i: (0, i, 0)),
            pl.BlockSpec((BR, D), lambda i: (i, 0)),
            pl.BlockSpec((BR, 16), lambda i: (i, 0)),
            pl.BlockSpec((1, D), lambda i: (0, 0)),
        ],
        out_specs=pl.BlockSpec((BR, D), lambda i: (i, 0)),
        out_shape=jax.ShapeDtypeStruct((N, D), jnp.float32),
    )(z, y, dinv16, b.reshape(1, D))


def kernel(x, edge_index, relations, W1, b1, g1, be1, W2, b2, g2, be2,
           W3, b3):
    del relations
    pad = E_PAD - E
    src = jnp.concatenate(
        [edge_index[0], jnp.zeros((pad,), jnp.int32)]).reshape(
            NW, 2, HGPT, CH)
    dst = jnp.concatenate(
        [edge_index[1], jnp.full((pad,), DUMMY, jnp.int32)]).reshape(
            NW, 2, HGPT, CH)
    zeros128 = jnp.zeros((RPT, D), jnp.float32)
    ones128 = jnp.ones((CH, D), jnp.float32)

    hist = _sc_deg(dst, zeros128, ones128)
    xw1 = _tc_mm(x, W1)
    dinv16, y1 = _tc_prep(hist, xw1)
    z1 = _sc_agg(y1, src, dst, zeros128)
    y2 = _tc_stage(z1, y1, dinv16, W2, b1, g1, be1)
    z2 = _sc_agg(y2, src, dst, zeros128)
    y3 = _tc_stage(z2, y2, dinv16, W3, b2, g2, be2)
    z3 = _sc_agg(y3, src, dst, zeros128)
    return _tc_final(z3, y3, dinv16, b3)


# speedup vs baseline: 17.8932x; 2.5991x over previous
import jax
import jax.numpy as jnp
from jax import lax
from jax.experimental import pallas as pl
from jax.experimental.pallas import tpu as pltpu
from jax.experimental.pallas import tpu_sc as plsc

N = 10000
D = 128
DH = 64
E = 320000

NC = 2
NS = 16
CH = 128
NH = 4
HGPT = 40
EPT = CH * NH * HGPT
E_PAD = NS * EPT
NZ = 10240
RPT = NZ // NS
NRS = 624
DUMMY = N

BR = 1000
BN_S = (1.0 + 1e-5) ** -0.5

_SC_PARAMS = pltpu.CompilerParams(use_tc_tiling_on_sc=False)


def _deg_body(dstr_hbm, zeros_hbm, ones_hbm, hist_out, hist_sh, idx_d,
              ones_v):
    c = lax.axis_index("c")
    s = lax.axis_index("s")
    r0 = s * RPT
    pltpu.sync_copy(zeros_hbm, hist_sh.at[pl.ds(r0, RPT)])
    pltpu.sync_copy(ones_hbm, ones_v)
    plsc.subcore_barrier()

    for j in range(NH // 2):
        pltpu.sync_copy(dstr_hbm.at[s, (NH // 2) * c + j], idx_d)

        def step(g, carry):
            pltpu.sync_copy(ones_v, hist_sh.at[idx_d.at[g]], add=True)
            return carry

        lax.fori_loop(0, HGPT, step, 0)

    plsc.subcore_barrier()
    pltpu.sync_copy(hist_sh.at[pl.ds(r0, RPT)],
                    hist_out.at[c, pl.ds(r0, RPT)])


def _sc_deg(dstr, zeros64, ones64):
    mesh = plsc.VectorSubcoreMesh(core_axis_name="c", subcore_axis_name="s")
    fn = pl.kernel(
        _deg_body,
        mesh=mesh,
        compiler_params=_SC_PARAMS,
        out_type=jax.ShapeDtypeStruct((NC, NZ, DH), jnp.float32),
        scratch_types=[
            pltpu.VMEM_SHARED((NZ, DH), jnp.float32),
            pltpu.VMEM((HGPT, CH), jnp.int32),
            pltpu.VMEM((CH, DH), jnp.float32),
        ],
    )
    return fn(dstr, zeros64, ones64)


def _agg_body(y2_hbm, srcr_hbm, dstr_hbm, zeros_hbm, z_out, y_sh, z_sh,
              idx_s, idx_d, rows0, rows1, sem0, sem1):
    c = lax.axis_index("c")
    s = lax.axis_index("s")
    r0 = s * RPT
    pltpu.sync_copy(zeros_hbm, z_sh.at[pl.ds(r0, RPT)])
    pltpu.sync_copy(y2_hbm.at[c, pl.ds(s * NRS, NRS)],
                    y_sh.at[pl.ds(s * NRS, NRS)])

    @pl.when(s == NS - 1)
    def _():
        pltpu.sync_copy(y2_hbm.at[c, pl.ds(NS * NRS, N - NS * NRS)],
                        y_sh.at[pl.ds(NS * NRS, N - NS * NRS)])

    plsc.subcore_barrier()

    for h in range(NH):
        pltpu.sync_copy(srcr_hbm.at[s, h], idx_s)
        pltpu.sync_copy(dstr_hbm.at[s, h], idx_d)
        pltpu.async_copy(y_sh.at[idx_s.at[0]], rows0, sem0)

        def pair(i, carry):
            g = 2 * i
            pltpu.async_copy(y_sh.at[idx_s.at[g + 1]], rows1, sem1)
            pltpu.make_async_copy(y_sh.at[idx_s.at[g]], rows0, sem0).wait()
            pltpu.sync_copy(rows0, z_sh.at[idx_d.at[g]], add=True)

            @pl.when(g + 2 < HGPT)
            def _():
                pltpu.async_copy(y_sh.at[idx_s.at[g + 2]], rows0, sem0)

            pltpu.make_async_copy(y_sh.at[idx_s.at[g + 1]], rows1,
                                  sem1).wait()
            pltpu.sync_copy(rows1, z_sh.at[idx_d.at[g + 1]], add=True)
            return carry

        lax.fori_loop(0, HGPT // 2, pair, 0)

    plsc.subcore_barrier()
    pltpu.sync_copy(z_sh.at[pl.ds(r0, RPT)], z_out.at[c, pl.ds(r0, RPT)])


def _sc_agg(y2, srcr, dstr, zeros64):
    mesh = plsc.VectorSubcoreMesh(core_axis_name="c", subcore_axis_name="s")
    fn = pl.kernel(
        _agg_body,
        mesh=mesh,
        compiler_params=_SC_PARAMS,
        out_type=jax.ShapeDtypeStruct((NC, NZ, DH), jnp.float32),
        scratch_types=[
            pltpu.VMEM_SHARED((N, DH), jnp.float32),
            pltpu.VMEM_SHARED((NZ, DH), jnp.float32),
            pltpu.VMEM((HGPT, CH), jnp.int32),
            pltpu.VMEM((HGPT, CH), jnp.int32),
            pltpu.VMEM((CH, DH), jnp.float32),
            pltpu.VMEM((CH, DH), jnp.float32),
            pltpu.SemaphoreType.DMA,
            pltpu.SemaphoreType.DMA,
        ],
    )
    return fn(y2, srcr, dstr, zeros64)


def _mm_body(x_ref, w_ref, o_ref):
    o_ref[...] = jnp.dot(x_ref[...], w_ref[...],
                         preferred_element_type=jnp.float32)


def _tc_mm(x, w):
    return pl.pallas_call(
        _mm_body,
        grid=(N // BR,),
        in_specs=[
            pl.BlockSpec((BR, D), lambda i: (i, 0)),
            pl.BlockSpec((D, D), lambda i: (0, 0)),
        ],
        out_specs=pl.BlockSpec((BR, D), lambda i: (i, 0)),
        out_shape=jax.ShapeDtypeStruct((N, D), jnp.float32),
    )(x, w)


def _split_cols(y):
    return jnp.stack([y[:, :DH], y[:, DH:]])


def _prep_body(hist_ref, xw_ref, dinv_ref, y2_ref):
    deg = hist_ref[0, :, 0:1] + hist_ref[1, :, 0:1] + 1.0
    dv = lax.rsqrt(deg)
    dinv_ref[...] = jnp.broadcast_to(dv, (BR, 16))
    y = xw_ref[...] * dv
    y2_ref[0] = y[:, :DH]
    y2_ref[1] = y[:, DH:]


def _tc_prep(hist, xw):
    return pl.pallas_call(
        _prep_body,
        grid=(N // BR,),
        in_specs=[
            pl.BlockSpec((NC, BR, DH), lambda i: (0, i, 0)),
            pl.BlockSpec((BR, D), lambda i: (i, 0)),
        ],
        out_specs=[
            pl.BlockSpec((BR, 16), lambda i: (i, 0)),
            pl.BlockSpec((NC, BR, DH), lambda i: (0, i, 0)),
        ],
        out_shape=[
            jax.ShapeDtypeStruct((N, 16), jnp.float32),
            jax.ShapeDtypeStruct((NC, N, DH), jnp.float32),
        ],
    )(hist, xw)


def _stage_body(z_ref, y_ref, dinv_ref, w_ref, b_ref, g_ref, be_ref,
                yo_ref):
    dv = dinv_ref[:, 0:1]
    zc = jnp.concatenate([z_ref[0], z_ref[1]], axis=1)
    yc = jnp.concatenate([y_ref[0], y_ref[1]], axis=1)
    t = (zc + yc) * dv + b_ref[...]
    h = jnp.maximum(t * (BN_S * g_ref[...]) + be_ref[...], 0.0)
    yn = jnp.dot(h, w_ref[...], preferred_element_type=jnp.float32) * dv
    yo_ref[0] = yn[:, :DH]
    yo_ref[1] = yn[:, DH:]


def _tc_stage(z, y2, dinv16, w_next, b, g, be):
    return pl.pallas_call(
        _stage_body,
        grid=(N // BR,),
        in_specs=[
            pl.BlockSpec((NC, BR, DH), lambda i: (0, i, 0)),
            pl.BlockSpec((NC, BR, DH), lambda i: (0, i, 0)),
            pl.BlockSpec((BR, 16), lambda i: (i, 0)),
            pl.BlockSpec((D, D), lambda i: (0, 0)),
            pl.BlockSpec((1, D), lambda i: (0, 0)),
            pl.BlockSpec((1, D), lambda i: (0, 0)),
            pl.BlockSpec((1, D), lambda i: (0, 0)),
        ],
        out_specs=pl.BlockSpec((NC, BR, DH), lambda i: (0, i, 0)),
        out_shape=jax.ShapeDtypeStruct((NC, N, DH), jnp.float32),
    )(z, y2, dinv16, w_next, b.reshape(1, D), g.reshape(1, D),
      be.reshape(1, D))


def _final_body(z_ref, y_ref, dinv_ref, b_ref, o_ref):
    dv = dinv_ref[:, 0:1]
    zc = jnp.concatenate([z_ref[0], z_ref[1]], axis=1)
    yc = jnp.concatenate([y_ref[0], y_ref[1]], axis=1)
    o = (zc + yc) * dv + b_ref[...]
    m = jnp.max(o, axis=1, keepdims=True)
    lse = jnp.log(jnp.sum(jnp.exp(o - m), axis=1, keepdims=True)) + m
    o_ref[...] = o - lse


def _tc_final(z, y2, dinv16, b):
    return pl.pallas_call(
        _final_body,
        grid=(N // BR,),
        in_specs=[
            pl.BlockSpec((NC, BR, DH), lambda i: (0, i, 0)),
            pl.BlockSpec((NC, BR, DH), lambda i: (0, i, 0)),
            pl.BlockSpec((BR, 16), lambda i: (i, 0)),
            pl.BlockSpec((1, D), lambda i: (0, 0)),
        ],
        out_specs=pl.BlockSpec((BR, D), lambda i: (i, 0)),
        out_shape=jax.ShapeDtypeStruct((N, D), jnp.float32),
    )(z, y2, dinv16, b.reshape(1, D))


def kernel(x, edge_index, relations, W1, b1, g1, be1, W2, b2, g2, be2,
           W3, b3):
    del relations
    pad = E_PAD - E
    src = jnp.concatenate(
        [edge_index[0], jnp.zeros((pad,), jnp.int32)]).reshape(
            NS, NH, HGPT, CH)
    dst = jnp.concatenate(
        [edge_index[1], jnp.full((pad,), DUMMY, jnp.int32)]).reshape(
            NS, NH, HGPT, CH)
    zeros64 = jnp.zeros((RPT, DH), jnp.float32)
    ones64 = jnp.ones((CH, DH), jnp.float32)

    hist = _sc_deg(dst, zeros64, ones64)
    xw1 = _tc_mm(x, W1)
    dinv16, y1 = _tc_prep(hist, xw1)
    z1 = _sc_agg(y1, src, dst, zeros64)
    y2 = _tc_stage(z1, y1, dinv16, W2, b1, g1, be1)
    z2 = _sc_agg(y2, src, dst, zeros64)
    y3 = _tc_stage(z2, y2, dinv16, W3, b2, g2, be2)
    z3 = _sc_agg(y3, src, dst, zeros64)
    return _tc_final(z3, y3, dinv16, b3)
